# Initial kernel scaffold; baseline (speedup 1.0000x reference)
import jax
import jax.numpy as jnp
from jax.experimental import pallas as pl


def kernel(u, l, pos_i, neg_i, emb_u, emb_i, emb_l, emb_ii, ui_rows, ui_cols, ui_vals, iu_rows, iu_cols, iu_vals, ij_rows, ij_cols, ij_vals, ji_rows, ji_cols, ji_vals):
    raise NotImplementedError("write your pallas kernel here")


# speedup vs baseline: 16.8565x; 16.8565x over previous
import functools

import jax
import jax.numpy as jnp
from jax import lax
from jax.experimental import pallas as pl
from jax.experimental.pallas import tpu as pltpu
from jax.experimental.pallas import tpu_sc as plsc

N_USERS = 100000
N_ITEMS = 50000
EMB = 32
NNZ_UI = 1600000
NNZ_II = 800000
B = 4096

NC = 2
NS = 16
NW = NC * NS
LANES = 16
GROUP = 128

_DUMMY_MASK = 8191


def _ceil_div(a, b):
    return (a + b - 1) // b


def _make_spmm(n_rows, nnz, ch):
    CH = ch
    CHE = CH * GROUP
    nr_groups = _ceil_div(nnz // GROUP, CH) * CH
    n_chunks = nr_groups // CH
    k_out = _ceil_div(n_chunks, NW)

    r0 = ((n_rows // 2 + 15) // 16) * 16
    r1 = n_rows - r0
    assert r0 % 8 == 0 and r1 % 8 == 0 and r1 > _DUMMY_MASK
    qa0 = _ceil_div(_ceil_div(r0, NS), 8) * 8
    qlast0 = r0 - (NS - 1) * qa0
    qa1 = _ceil_div(_ceil_div(r1, NS), 8) * 8
    qlast1 = r1 - (NS - 1) * qa1
    assert qlast0 > 0 and qlast1 > 0

    def body(rows_hbm, cols_hbm, vals_hbm, x_hbm, out_hbm,
             ebuf_r, ebuf_c, ebuf_v, rloc, gbuf, acc, sem_g, sem_s):
        cc = lax.axis_index("c")
        ss = lax.axis_index("s")
        w = ss * NC + cc
        lo = cc * r0
        hi = lo + jnp.where(cc == 0, r0, r1)
        i16 = lax.iota(jnp.int32, 16)
        z16 = jnp.zeros((16,), jnp.float32)

        @pl.loop(0, CHE, unroll=8)
        def _zero_gbuf(e):
            gbuf[e, pl.ds(0, 16)] = z16
            gbuf[e, pl.ds(16, 16)] = z16

        def _zero_stripe(qa, sz_stripe):
            off = 0
            while off < sz_stripe:
                sz = min(CHE, sz_stripe - off)
                pltpu.sync_copy(gbuf.at[pl.ds(0, sz)],
                                acc.at[pl.ds(ss * qa + off, sz)])
                off += sz

        for cc_v, qa, qlast in ((0, qa0, qlast0), (1, qa1, qlast1)):
            @pl.when((cc == cc_v) & (ss < NS - 1))
            def _(qa=qa):
                _zero_stripe(qa, qa)

            @pl.when((cc == cc_v) & (ss == NS - 1))
            def _(qa=qa, qlast=qlast):
                _zero_stripe(qa, qlast)

        plsc.subcore_barrier()

        @pl.loop(0, k_out)
        def _chunk(kk):
            c = kk * NW + w

            @pl.when(c < n_chunks)
            def _():
                gbase = c * CH
                pltpu.sync_copy(rows_hbm.at[pl.ds(gbase, CH)], ebuf_r)
                pltpu.sync_copy(cols_hbm.at[pl.ds(gbase, CH)], ebuf_c)
                pltpu.sync_copy(vals_hbm.at[pl.ds(gbase, CH)], ebuf_v)

                @pl.loop(0, CH)
                def _prep(g):
                    for j in range(GROUP // 16):
                        rg = ebuf_r[g, pl.ds(j * 16, 16)]
                        vv = ebuf_v[g, pl.ds(j * 16, 16)]
                        m = (rg >= lo) & (rg < hi)
                        dummy = ((g * 8 + j) * 16 + i16) & _DUMMY_MASK
                        rloc[g, pl.ds(j * 16, 16)] = jnp.where(m, rg - lo, dummy)
                        ebuf_v[g, pl.ds(j * 16, 16)] = jnp.where(m, vv, 0.0)

                descs = [
                    pltpu.async_copy(
                        x_hbm.at[ebuf_c.at[g]],
                        gbuf.at[pl.ds(g * GROUP, GROUP)], sem_g)
                    for g in range(CH)
                ]
                for d in descs:
                    d.wait()

                @pl.loop(0, CH)
                def _scale(g):
                    @pl.loop(0, GROUP // 16)
                    def _scale_grp(jb):
                        vv = ebuf_v[g, pl.ds(jb * 16, 16)]
                        eb = g * GROUP + jb * 16
                        for t in range(16):
                            spl = jnp.full((16,), vv[t], jnp.float32)
                            gbuf[eb + t, pl.ds(0, 16)] = (
                                gbuf[eb + t, pl.ds(0, 16)] * spl)
                            gbuf[eb + t, pl.ds(16, 16)] = (
                                gbuf[eb + t, pl.ds(16, 16)] * spl)

                descs2 = [
                    pltpu.async_copy(
                        gbuf.at[pl.ds(g * GROUP, GROUP)],
                        acc.at[rloc.at[g]], sem_s, add=True)
                    for g in range(CH)
                ]
                for d in descs2:
                    d.wait()

        plsc.subcore_barrier()

        for cc_v, qa, qlast, base in ((0, qa0, qlast0, 0), (1, qa1, qlast1, r0)):
            @pl.when((cc == cc_v) & (ss < NS - 1))
            def _(qa=qa, base=base):
                pltpu.sync_copy(acc.at[pl.ds(ss * qa, qa)],
                                out_hbm.at[pl.ds(base + ss * qa, qa)])

            @pl.when((cc == cc_v) & (ss == NS - 1))
            def _(qa=qa, qlast=qlast, base=base):
                pltpu.sync_copy(acc.at[pl.ds(ss * qa, qlast)],
                                out_hbm.at[pl.ds(base + ss * qa, qlast)])

    return pl.kernel(
        body,
        out_type=jax.ShapeDtypeStruct((n_rows, EMB), jnp.float32),
        mesh=plsc.VectorSubcoreMesh(core_axis_name="c", subcore_axis_name="s"),
        compiler_params=pltpu.CompilerParams(use_tc_tiling_on_sc=False),
        scratch_types=[
            pltpu.VMEM((CH, GROUP), jnp.int32),
            pltpu.VMEM((CH, GROUP), jnp.int32),
            pltpu.VMEM((CH, GROUP), jnp.float32),
            pltpu.VMEM((CH, GROUP), jnp.int32),
            pltpu.VMEM((CHE, EMB), jnp.float32),
            pltpu.VMEM_SHARED((r0, EMB), jnp.float32),
            pltpu.SemaphoreType.DMA,
            pltpu.SemaphoreType.DMA,
        ],
    )


CH_USERS = 5
CH_ITEMS = 16

_spmm_users = _make_spmm(N_USERS, NNZ_UI, CH_USERS)
_spmm_items_big = _make_spmm(N_ITEMS, NNZ_UI, CH_ITEMS)
_spmm_items_small = _make_spmm(N_ITEMS, NNZ_II, CH_ITEMS)


def _gather6_body(t_uu, t_ui, t_il, t_ii, u2d, l2d, p2d, n2d,
                  o_u, o_pu, o_nu, o_l, o_pl, o_nl, ibuf, gb, sem):
    cc = lax.axis_index("c")
    ss = lax.axis_index("s")
    w = ss * NC + cc
    for tbl, idx2d, out in ((t_uu, u2d, o_u), (t_ui, p2d, o_pu),
                            (t_ui, n2d, o_nu), (t_il, l2d, o_l),
                            (t_ii, p2d, o_pl), (t_ii, n2d, o_nl)):
        pltpu.sync_copy(idx2d.at[w], ibuf)
        pltpu.async_copy(tbl.at[ibuf.at[0]], gb, sem).wait()
        pltpu.sync_copy(gb, out.at[pl.ds(w * GROUP, GROUP)])


_gather6 = pl.kernel(
    _gather6_body,
    out_type=[jax.ShapeDtypeStruct((B, EMB), jnp.float32)] * 6,
    mesh=plsc.VectorSubcoreMesh(core_axis_name="c", subcore_axis_name="s"),
    compiler_params=pltpu.CompilerParams(use_tc_tiling_on_sc=False),
    scratch_types=[
        pltpu.VMEM((1, GROUP), jnp.int32),
        pltpu.VMEM((GROUP, EMB), jnp.float32),
        pltpu.SemaphoreType.DMA,
    ],
)


def _l2n(v):
    return v / jnp.maximum(jnp.sqrt(jnp.sum(v * v, axis=1, keepdims=True)),
                           1e-12)


def _l2norm_body(x_ref, o_ref):
    o_ref[...] = _l2n(x_ref[...])


def _avg3_body(e_ref, a_ref, b_ref, o_ref):
    o_ref[...] = (e_ref[...] + _l2n(a_ref[...]) + _l2n(b_ref[...])) * (1.0 / 3.0)


def _tc_rows_call(body, n_inputs, n_rows, block_rows=2000):
    grid = n_rows // block_rows
    spec = pl.BlockSpec((block_rows, EMB), lambda i: (i, 0))
    return pl.pallas_call(
        body,
        out_shape=jax.ShapeDtypeStruct((n_rows, EMB), jnp.float32),
        grid=(grid,),
        in_specs=[spec] * n_inputs,
        out_specs=spec,
    )


_l2norm_u = _tc_rows_call(_l2norm_body, 1, N_USERS)
_l2norm_i = _tc_rows_call(_l2norm_body, 1, N_ITEMS)
_avg3_u = _tc_rows_call(_avg3_body, 3, N_USERS)
_avg3_i = _tc_rows_call(_avg3_body, 3, N_ITEMS)


def _loss_body(ur, pu, nu, lr, pli, nli, o_ref):
    u = ur[...]
    lv = lr[...]
    pos = jnp.sum(u * pu[...] + lv * pli[...], axis=1)
    neg = jnp.sum(u * nu[...] + lv * nli[...], axis=1)
    d = pos - neg
    sp = jnp.maximum(-d, 0.0) + jnp.log(1.0 + jnp.exp(-jnp.abs(d)))
    o_ref[...] = jnp.mean(sp).reshape(1, 1)


_loss_tc = pl.pallas_call(
    _loss_body,
    out_shape=jax.ShapeDtypeStruct((1, 1), jnp.float32),
    in_specs=[pl.BlockSpec((B, EMB), lambda: (0, 0))] * 6,
    out_specs=pl.BlockSpec((1, 1), lambda: (0, 0)),
)


def kernel(u, l, pos_i, neg_i, emb_u, emb_i, emb_l, emb_ii,
           ui_rows, ui_cols, ui_vals, iu_rows, iu_cols, iu_vals,
           ij_rows, ij_cols, ij_vals, ji_rows, ji_cols, ji_vals):
    def e2d(a, ch):
        nr = _ceil_div(a.shape[0] // GROUP, ch) * ch
        pad = nr * GROUP - a.shape[0]
        if pad:
            a = jnp.pad(a, (0, pad))
        return a.reshape(nr, GROUP)

    ui = tuple(e2d(a, CH_USERS) for a in (ui_rows, ui_cols, ui_vals))
    iu = tuple(e2d(a, CH_ITEMS) for a in (iu_rows, iu_cols, iu_vals))
    ij = tuple(e2d(a, CH_ITEMS) for a in (ij_rows, ij_cols, ij_vals))
    ji = tuple(e2d(a, CH_ITEMS) for a in (ji_rows, ji_cols, ji_vals))

    s1 = _spmm_users(*ui, emb_i)
    s1p = _spmm_items_big(*iu, emb_u)
    tn1 = _l2norm_i(s1p)
    tn2 = _l2norm_u(s1)
    s2 = _spmm_users(*ui, tn1)
    s2p = _spmm_items_big(*iu, tn2)
    ui_rep_u = _avg3_u(emb_u, s1, s2)
    ui_rep_i = _avg3_i(emb_i, s1p, s2p)

    c1 = _spmm_items_small(*ij, emb_ii)
    c2 = _spmm_items_small(*ij, c1)
    d1 = _spmm_items_small(*ji, emb_l)
    d2 = _spmm_items_small(*ji, d1)
    ii_rep_l = _avg3_i(emb_l, c1, c2)
    ii_rep_i = _avg3_i(emb_ii, d1, d2)

    def b2d(a):
        return a.reshape(NW, 1, GROUP)

    g_u, g_pu, g_nu, g_l, g_pl, g_nl = _gather6(
        ui_rep_u, ui_rep_i, ii_rep_l, ii_rep_i,
        b2d(u), b2d(l), b2d(pos_i), b2d(neg_i))

    loss = _loss_tc(g_u, g_pu, g_nu, g_l, g_pl, g_nl)
    return loss.reshape(())
